# baseline (speedup 1.0000x reference)
import jax
import jax.numpy as jnp
from jax import lax
from jax.experimental import pallas as pl
from jax.experimental.pallas import tpu as pltpu
from jax.experimental.pallas import tpu_sc as plsc

_B = 8
_C = 128
_H = 224
_W = 224
_WIN = 64
_NCORE = 2
_NSUB = 16
_QCH = _C // 4
_RPW = _QCH * 8
_NEG = -5e29


def _sc_gather(q_ref, par_ref, out_ref, par_v, idx_a, idx_b, rows_v,
               comp_v, sem):
    wid = lax.axis_index("c") * _NSUB + lax.axis_index("s")
    b = wid // 4
    quad = wid % 4
    pltpu.sync_copy(par_ref, par_v)
    bsplat = jnp.full((16,), b, jnp.int32)
    sr = plsc.load_gather(par_v, [bsplat])
    delta = plsc.load_gather(par_v, [bsplat + 8])
    cb = plsc.load_gather(par_v, [bsplat + 16])

    ch0 = b * _C + quad * _QCH
    lane = lax.iota(jnp.int32, 16)
    for m in range(8):
        e = m * 16 + lane
        idx_a[pl.ds(m * 16, 16)] = (ch0 + e // 8) * _H + sr + e % 8
    for m in range(8):
        e = 128 + m * 16 + lane
        idx_b[pl.ds(m * 16, 16)] = (ch0 + e // 8) * _H + sr + e % 8
    pltpu.async_copy(q_ref.at[idx_a], rows_v.at[pl.ds(0, 128)], sem).wait()
    pltpu.async_copy(q_ref.at[idx_b], rows_v.at[pl.ds(128, 128)], sem).wait()

    zero = jnp.zeros((16,), jnp.int32)
    seven = jnp.full((16,), 7, jnp.int32)
    wmax = jnp.full((16,), _W - 1, jnp.int32)
    for i in range(_QCH):
        for m in range(4):
            w = m * 16 + lane
            k = w // 8
            j = w % 8
            lr = jnp.minimum(jnp.maximum(k + delta, zero), seven)
            col = jnp.minimum(jnp.maximum(cb + j, zero), wmax)
            vals = plsc.load_gather(rows_v, [i * 8 + lr, col])
            plsc.store_scatter(comp_v, [jnp.full((16,), i, jnp.int32),
                                        m * 16 + lane], vals)
    pltpu.sync_copy(comp_v, out_ref.at[pl.ds(b * _C + quad * _QCH, _QCH)])


def _dot3(a, b, dims):
    ah = a.astype(jnp.bfloat16)
    al = (a - ah.astype(jnp.float32)).astype(jnp.bfloat16)
    bh = b.astype(jnp.bfloat16)
    bl = (b - bh.astype(jnp.float32)).astype(jnp.bfloat16)
    f = jnp.float32
    return (lax.dot_general(ah, bh, dims, preferred_element_type=f) +
            (lax.dot_general(ah, bl, dims, preferred_element_type=f) +
             lax.dot_general(al, bh, dims, preferred_element_type=f)))


_MM = (((1,), (0,)), ((), ()))


def _tc_attn(pat_ref, ct_ref, wa_ref, bias_ref, out_ref):
    v_all = _dot3(ct_ref[...], wa_ref[...], _MM)
    v_t = v_all.T
    for b in range(_B):
        g = pat_ref[b * _C:(b + 1) * _C, :]
        s = jnp.sum(g * v_t[:, b:b + 1], axis=0, keepdims=True)
        s = s + bias_ref[b:b + 1, :]
        m = jnp.max(s)
        e = jnp.exp(s - m)
        wgt = e / jnp.sum(e)
        out_ref[:, b:b + 1] = jnp.sum(g * wgt, axis=1, keepdims=True)


def kernel(q_i, c_t, w_a, w_p):
    f32 = jnp.float32
    loc = jax.nn.sigmoid(c_t @ w_p.T)
    p_x = loc[:, 0] * (_H + 1 - 2)
    p_y = loc[:, 1] * (_W + 1 - 2)
    px_r = jnp.round(p_x).astype(jnp.int32)
    py_r = jnp.round(p_y).astype(jnp.int32)
    sr = jnp.clip(px_r - 4, 0, _H - 8)
    params = jnp.concatenate([sr, (px_r - 4) - sr, py_r - 4,
                              jnp.zeros((8,), jnp.int32)])

    k = jnp.arange(8)
    u = px_r[:, None] - 4 + k[None, :]
    br = jnp.where((u >= 0) & (u <= _H - 1),
                   -2.0 * ((u.astype(f32) - p_x[:, None]) / 4.0) ** 2, _NEG)
    v = py_r[:, None] - 4 + k[None, :]
    bc = jnp.where((v >= 0) & (v <= _W - 1),
                   -2.0 * ((v.astype(f32) - p_y[:, None]) / 4.0) ** 2, _NEG)
    bias = (br[:, :, None] + bc[:, None, :]).reshape(_B, _WIN)

    q_rows = q_i.reshape(_B * _C * _H, _W)

    patches = pl.kernel(
        _sc_gather,
        out_type=jax.ShapeDtypeStruct((_B * _C, _WIN), f32),
        mesh=plsc.VectorSubcoreMesh(core_axis_name="c", subcore_axis_name="s",
                                    num_cores=_NCORE, num_subcores=_NSUB),
        scratch_types=[
            pltpu.VMEM((32,), jnp.int32),
            pltpu.VMEM((128,), jnp.int32),
            pltpu.VMEM((128,), jnp.int32),
            pltpu.VMEM((_RPW, _W), f32),
            pltpu.VMEM((_QCH, _WIN), f32),
            pltpu.SemaphoreType.DMA,
        ],
        compiler_params=pltpu.CompilerParams(needs_layout_passes=False,
                                             use_tc_tiling_on_sc=False),
    )(q_rows, params)

    out_t = pl.pallas_call(
        _tc_attn,
        in_specs=[
            pl.BlockSpec(memory_space=pltpu.MemorySpace.VMEM),
            pl.BlockSpec(memory_space=pltpu.MemorySpace.VMEM),
            pl.BlockSpec(memory_space=pltpu.MemorySpace.VMEM),
            pl.BlockSpec(memory_space=pltpu.MemorySpace.VMEM),
        ],
        out_specs=pl.BlockSpec(memory_space=pltpu.MemorySpace.VMEM),
        out_shape=jax.ShapeDtypeStruct((_C, _B), f32),
    )(patches, c_t, w_a, bias)
    return out_t.T


# speedup vs baseline: 2.0165x; 2.0165x over previous
import jax
import jax.numpy as jnp
from jax import lax
from jax.experimental import pallas as pl
from jax.experimental.pallas import tpu as pltpu

_B = 8
_C = 128
_H = 224
_W = 224
_ROWS = 16
_LW = 256
_NEG = -5e29


def _dot3(a, b):
    ah = a.astype(jnp.bfloat16)
    al = (a - ah.astype(jnp.float32)).astype(jnp.bfloat16)
    bh = b.astype(jnp.bfloat16)
    bl = (b - bh.astype(jnp.float32)).astype(jnp.bfloat16)
    f = jnp.float32
    return (jnp.dot(ah, bh, preferred_element_type=f) +
            (jnp.dot(ah, bl, preferred_element_type=f) +
             jnp.dot(al, bh, preferred_element_type=f)))


def _attn_kernel(sr8_ref, c1_ref, q_ref, ct_ref, wa_ref, bias_ref,
                 out_ref, patch_ref, sems):
    def dma(b, tile):
        c0 = 0 if tile == 0 else pl.multiple_of(c1_ref[0], 128)
        return pltpu.make_async_copy(
            q_ref.at[b, :, pl.ds(pl.multiple_of(sr8_ref[b], 8), _ROWS),
                     pl.ds(c0, 128)],
            patch_ref.at[b, :, :, pl.ds(128 * tile, 128)],
            sems.at[b],
        )

    for b in range(_B):
        dma(b, 0).start()
        dma(b, 1).start()
    v_all = _dot3(ct_ref[...], wa_ref[...])
    v_t = v_all.T

    for b in range(_B):
        dma(b, 0).wait()
        dma(b, 1).wait()
        vcol = v_t[:, b:b + 1]
        rows = []
        scs = []
        for i in range(_ROWS):
            brow = bias_ref[b, i:i + 1, :]
            row = patch_ref[b, :, i, :]
            row = jnp.where(brow > -1e28, row, 0.0)
            rows.append(row)
            scs.append(jnp.sum(row * vcol, axis=0, keepdims=True) + brow)
        s = jnp.concatenate(scs, axis=0)
        m = jnp.max(s)
        e = jnp.exp(s - m)
        wgt = e / jnp.sum(e)
        acc = None
        for i in range(_ROWS):
            t = jnp.sum(rows[i] * wgt[i:i + 1, :], axis=1, keepdims=True)
            acc = t if acc is None else acc + t
        out_ref[:, b:b + 1] = acc


def kernel(q_i, c_t, w_a, w_p):
    f32 = jnp.float32
    loc = jax.nn.sigmoid(c_t @ w_p.T)
    p_x = loc[:, 0] * (_H + 1 - 2)
    p_y = loc[:, 1] * (_W + 1 - 2)
    px_r = jnp.round(p_x).astype(jnp.int32)
    py_r = jnp.round(p_y).astype(jnp.int32)
    sr = jnp.clip(px_r - 4, 0, _H - 8)
    sr8 = jnp.minimum((sr // 8) * 8, _H - _ROWS)
    c1 = jnp.full((1,), 128, jnp.int32)

    u = sr8[:, None] + jnp.arange(_ROWS)[None, :]
    mr = (u >= px_r[:, None] - 4) & (u <= px_r[:, None] + 3)
    br = jnp.where(mr, -2.0 * ((u.astype(f32) - p_x[:, None]) / 4.0) ** 2,
                   _NEG)
    w = jnp.arange(_LW)[None, :]
    mc = ((w >= py_r[:, None] - 4) & (w <= py_r[:, None] + 3) &
          (w <= _W - 1))
    bc = jnp.where(mc, -2.0 * ((w.astype(f32) - p_y[:, None]) / 4.0) ** 2,
                   _NEG)
    bias = br[:, :, None] + bc[:, None, :]

    grid_spec = pltpu.PrefetchScalarGridSpec(
        num_scalar_prefetch=2,
        grid=(1,),
        in_specs=[
            pl.BlockSpec(memory_space=pltpu.MemorySpace.HBM),
            pl.BlockSpec(memory_space=pltpu.MemorySpace.VMEM),
            pl.BlockSpec(memory_space=pltpu.MemorySpace.VMEM),
            pl.BlockSpec(memory_space=pltpu.MemorySpace.VMEM),
        ],
        out_specs=pl.BlockSpec(memory_space=pltpu.MemorySpace.VMEM),
        scratch_shapes=[
            pltpu.VMEM((_B, _C, _ROWS, _LW), f32),
            pltpu.SemaphoreType.DMA((_B,)),
        ],
    )
    out_t = pl.pallas_call(
        _attn_kernel,
        grid_spec=grid_spec,
        out_shape=jax.ShapeDtypeStruct((_C, _B), f32),
    )(sr8, c1, q_i, c_t, w_a, bias)
    return out_t.T
